# baseline (speedup 1.0000x reference)
import functools

import jax
import jax.numpy as jnp
from jax import lax
from jax.experimental import pallas as pl
from jax.experimental.pallas import tpu as pltpu
from jax.experimental.pallas import tpu_sc as plsc

_VS = 0.02
_G = 51
_V = _G ** 3
_D = 2048

_NC = 2
_NS = 16

_B = 4
_N = 200000
_NPT = ((_N + _NS * 128 - 1) // (_NS * 128)) * 128
_NPAD = _NS * _NPT
_DS = _D // _NS
_NROW = 8
_PB = 4096
_NBK = _NPAD // _PB
_PAD_VAL = 1e30


def _min_body(x_ref, y_ref, z_ref, o_ref):
  j = pl.program_id(1)

  @pl.when(j == 0)
  def _():
    o_ref[...] = jnp.full_like(o_ref, _PAD_VAL)

  def red(ref):
    return jnp.min(ref[0, 0].reshape(_PB // 128, 128), axis=0)

  m = jnp.stack([red(x_ref), red(y_ref), red(z_ref)])
  o_ref[...] = jnp.minimum(o_ref[...], m[None])


def _mins128(xc, yc, zc):
  return pl.pallas_call(
      _min_body,
      grid=(_B, _NBK),
      in_specs=[pl.BlockSpec((1, 1, _PB),
                             lambda b, j: (b * _NBK + j, 0, 0))] * 3,
      out_specs=pl.BlockSpec((1, 3, 128), lambda b, j: (b, 0, 0)),
      out_shape=jax.ShapeDtypeStruct((_B, 3, 128), jnp.float32),
      compiler_params=pltpu.CompilerParams(
          dimension_semantics=("arbitrary", "arbitrary")),
  )(xc, yc, zc)


def _rows_body(x_ref, y_ref, z_ref, m_ref, rowsa_ref, rowsb_ref, fl_ref):
  j = pl.program_id(1)
  m0 = jnp.min(m_ref[0, 0])
  m1 = jnp.min(m_ref[0, 1])
  m2 = jnp.min(m_ref[0, 2])
  shape = (_PB // 128, 128)
  gidx = (j * _PB
          + lax.broadcasted_iota(jnp.int32, shape, 0) * 128
          + lax.broadcasted_iota(jnp.int32, shape, 1))
  valid = gidx < _N
  px = x_ref[0, 0].reshape(shape)
  py = y_ref[0, 0].reshape(shape)
  pz = z_ref[0, 0].reshape(shape)
  vs = jnp.float32(_VS)

  def vox(p, m):
    pm = jnp.where(valid, p, m)
    return jnp.clip(((pm - m) / vs).astype(jnp.int32), 0, _G - 1)

  fl = (vox(px, m0) * _G + vox(py, m1)) * _G + vox(pz, m2)
  px = jnp.where(valid, px, 0.0)
  py = jnp.where(valid, py, 0.0)
  pz = jnp.where(valid, pz, 0.0)
  cnt = valid.astype(jnp.float32)
  z0 = jnp.zeros_like(cnt)
  va = (cnt, px, py, pz, px * px, px * py, px * pz, py * py)
  vb = (py * pz, pz * pz)
  for v in range(_NROW):
    rowsa_ref[0, v] = va[v]
  for v in range(2):
    rowsb_ref[0, v] = vb[v]
  fl_ref[0, 0] = fl.reshape(_PB)


def _rows_fl(xc, yc, zc, mins):
  return pl.pallas_call(
      _rows_body,
      grid=(_B, _NBK),
      in_specs=[pl.BlockSpec((1, 1, _PB),
                             lambda b, j: (b * _NBK + j, 0, 0))] * 3
      + [pl.BlockSpec((1, 3, 128), lambda b, j: (b, 0, 0))],
      out_specs=[
          pl.BlockSpec((1, _NROW, _PB // 128, 128),
                       lambda b, j: (b * _NBK + j, 0, 0, 0)),
          pl.BlockSpec((1, 2, _PB // 128, 128),
                       lambda b, j: (b * _NBK + j, 0, 0, 0)),
          pl.BlockSpec((1, 1, _PB), lambda b, j: (b * _NBK + j, 0, 0)),
      ],
      out_shape=[
          jax.ShapeDtypeStruct((_B * _NBK, _NROW, _PB // 128, 128),
                               jnp.float32),
          jax.ShapeDtypeStruct((_B * _NBK, 2, _PB // 128, 128),
                               jnp.float32),
          jax.ShapeDtypeStruct((_B * _NBK, 1, _PB), jnp.int32),
      ],
      compiler_params=pltpu.CompilerParams(
          dimension_semantics=("parallel", "parallel")),
  )(xc, yc, zc, mins)


def _sc_body(rowsa_hbm, rowsb_hbm, fl_hbm, gidx_hbm, zr_hbm,
             rawa_hbm, rawb_hbm,
             table_sh, flv, rowsv, sidxv, sflv, zrows, grows, sem):
  c = lax.axis_index("c")
  s = lax.axis_index("s")
  pltpu.sync_copy(zr_hbm, zrows)

  def do_all(rows_hbm, raw_hbm):
    for b in range(_B):
      rowbase = b * _NPAD + s * _NPT
      obase = b * _D + s * _DS

      pltpu.sync_copy(gidx_hbm.at[pl.ds(obase, _DS)], sidxv)
      pltpu.async_copy(fl_hbm.at[sidxv], sflv, sem).wait()
      plsc.subcore_barrier()
      pltpu.async_copy(zrows, table_sh.at[sflv], sem).wait()
      plsc.subcore_barrier()

      def chunk(k, _):
        off = rowbase + k * 128
        pltpu.sync_copy(fl_hbm.at[pl.ds(off, 128)], flv)
        pltpu.sync_copy(rows_hbm.at[pl.ds(off, 128)], rowsv)
        pltpu.sync_copy(rowsv, table_sh.at[flv], add=True)
        return 0

      lax.fori_loop(0, _NPT // 128, chunk, 0)
      plsc.subcore_barrier()
      pltpu.async_copy(table_sh.at[sflv], grows, sem).wait()
      pltpu.sync_copy(grows, raw_hbm.at[pl.ds(obase, _DS)])

  @pl.when(c == 0)
  def _():
    do_all(rowsa_hbm, rawa_hbm)

  @pl.when(c == 1)
  def _():
    do_all(rowsb_hbm, rawb_hbm)


def _sc_bin(rowsa, rowsb, fl1d, gidx):
  zr = jnp.zeros((_DS, _NROW), jnp.float32)
  run = pl.kernel(
      _sc_body,
      out_type=(jax.ShapeDtypeStruct((_B * _D, _NROW), jnp.float32),
                jax.ShapeDtypeStruct((_B * _D, _NROW), jnp.float32)),
      mesh=plsc.VectorSubcoreMesh(core_axis_name="c", subcore_axis_name="s",
                                  num_cores=_NC, num_subcores=_NS),
      scratch_types=[
          pltpu.VMEM_SHARED((_V, _NROW), jnp.float32),
          pltpu.VMEM((128,), jnp.int32),
          pltpu.VMEM((128, _NROW), jnp.float32),
          pltpu.VMEM((_DS,), jnp.int32),
          pltpu.VMEM((_DS,), jnp.int32),
          pltpu.VMEM((_DS, _NROW), jnp.float32),
          pltpu.VMEM((_DS, _NROW), jnp.float32),
          pltpu.SemaphoreType.DMA,
      ],
      compiler_params=pltpu.CompilerParams(use_tc_tiling_on_sc=False),
  )
  return run(rowsa, rowsb, fl1d, gidx, zr)


def _fin_body(ra_ref, rb_ref, o_ref):
  cnt = ra_ref[0, 0]
  safe = jnp.maximum(cnt, 1.0)
  mx = ra_ref[0, 1] / safe
  my = ra_ref[0, 2] / safe
  mz = ra_ref[0, 3] / safe
  cxx = ra_ref[0, 4] / safe - mx * mx
  cxy = ra_ref[0, 5] / safe - mx * my
  cxz = ra_ref[0, 6] / safe - mx * mz
  cyy = ra_ref[0, 7] / safe - my * my
  cyz = rb_ref[0, 0] / safe - my * mz
  czz = rb_ref[0, 1] / safe - mz * mz
  o_ref[0] = jnp.stack(
      [mx, my, mz, cxx, cxy, cxz, cxy, cyy, cyz, cxz, cyz, czz])


def _finalize(rawa_t, rawb_t):
  return pl.pallas_call(
      _fin_body,
      grid=(_B,),
      in_specs=[pl.BlockSpec((1, _NROW, _D), lambda b: (b, 0, 0))] * 2,
      out_specs=pl.BlockSpec((1, 12, _D), lambda b: (b, 0, 0)),
      out_shape=jax.ShapeDtypeStruct((_B, 12, _D), jnp.float32),
  )(rawa_t, rawb_t)


@jax.jit
def kernel(x):
  B, N, _ = x.shape
  skey = jax.random.key(42)
  sampled_idx = jax.random.randint(skey, (B, _D), 0, N)
  gidx = (sampled_idx.astype(jnp.int32)
          + jnp.arange(B, dtype=jnp.int32)[:, None] * _NPAD).reshape(B * _D)

  xt = x.transpose(0, 2, 1)
  xt = jnp.pad(xt, ((0, 0), (0, 0), (0, _NPAD - N)),
               constant_values=_PAD_VAL)
  xc = xt[:, 0].reshape(_B * _NBK, 1, _PB)
  yc = xt[:, 1].reshape(_B * _NBK, 1, _PB)
  zc = xt[:, 2].reshape(_B * _NBK, 1, _PB)

  mins = _mins128(xc, yc, zc)
  rowsa, rowsb, fl = _rows_fl(xc, yc, zc, mins)
  rowsa = rowsa.transpose(0, 2, 3, 1).reshape(B * _NPAD, _NROW)
  rowsb = jnp.pad(rowsb.transpose(0, 2, 3, 1).reshape(B * _NPAD, 2),
                  ((0, 0), (0, _NROW - 2)))
  rawa, rawb = _sc_bin(rowsa, rowsb, fl.reshape(B * _NPAD), gidx)
  out = _finalize(rawa.reshape(B, _D, _NROW).transpose(0, 2, 1),
                  rawb.reshape(B, _D, _NROW).transpose(0, 2, 1))
  return out.transpose(0, 2, 1)


# speedup vs baseline: 2.4912x; 2.4912x over previous
import functools

import jax
import jax.numpy as jnp
from jax import lax
from jax.experimental import pallas as pl
from jax.experimental.pallas import tpu as pltpu
from jax.experimental.pallas import tpu_sc as plsc

_VS = 0.02
_G = 51
_V = _G ** 3
_D = 2048

_NC = 2
_NS = 16

_B = 4
_N = 200000
_NPT = ((_N + _NS * 128 - 1) // (_NS * 128)) * 128
_NPAD = _NS * _NPT
_DS = _D // _NS
_NROW = 8
_PB = 4096
_NBK = _NPAD // _PB
_PAD_VAL = 1e30


def _min_body(x_ref, y_ref, z_ref, o_ref):
  j = pl.program_id(1)

  @pl.when(j == 0)
  def _():
    o_ref[...] = jnp.full_like(o_ref, _PAD_VAL)

  def red(ref):
    return jnp.min(ref[0, 0].reshape(_PB // 128, 128), axis=0)

  m = jnp.stack([red(x_ref), red(y_ref), red(z_ref)])
  o_ref[...] = jnp.minimum(o_ref[...], m[None])


def _mins128(xc, yc, zc):
  return pl.pallas_call(
      _min_body,
      grid=(_B, _NBK),
      in_specs=[pl.BlockSpec((1, 1, _PB),
                             lambda b, j: (b * _NBK + j, 0, 0))] * 3,
      out_specs=pl.BlockSpec((1, 3, 128), lambda b, j: (b, 0, 0)),
      out_shape=jax.ShapeDtypeStruct((_B, 3, 128), jnp.float32),
      compiler_params=pltpu.CompilerParams(
          dimension_semantics=("arbitrary", "arbitrary")),
  )(xc, yc, zc)


def _rows_body(x_ref, y_ref, z_ref, m_ref, rowsa_ref, rowsb_ref, fl_ref,
               fl1_ref):
  j = pl.program_id(1)
  m0 = jnp.min(m_ref[0, 0])
  m1 = jnp.min(m_ref[0, 1])
  m2 = jnp.min(m_ref[0, 2])
  shape = (_PB // 128, 128)
  gidx = (j * _PB
          + lax.broadcasted_iota(jnp.int32, shape, 0) * 128
          + lax.broadcasted_iota(jnp.int32, shape, 1))
  valid = gidx < _N
  px = x_ref[0, 0].reshape(shape)
  py = y_ref[0, 0].reshape(shape)
  pz = z_ref[0, 0].reshape(shape)
  vs = jnp.float32(_VS)

  def vox(p, m):
    pm = jnp.where(valid, p, m)
    return jnp.clip(((pm - m) / vs).astype(jnp.int32), 0, _G - 1)

  fl = (vox(px, m0) * _G + vox(py, m1)) * _G + vox(pz, m2)
  px = jnp.where(valid, px, 0.0)
  py = jnp.where(valid, py, 0.0)
  pz = jnp.where(valid, pz, 0.0)
  cnt = valid.astype(jnp.float32)
  z0 = jnp.zeros_like(cnt)
  va = (cnt, px, py, pz, px * px, px * py, px * pz, py * py)
  vb = (py * pz, pz * pz, z0, z0, z0, z0, z0, z0)
  for v in range(_NROW):
    rowsa_ref[0, v] = va[v]
    rowsb_ref[0, v] = vb[v]
  fl_ref[0, 0] = fl.reshape(_PB)
  fl1_ref[0, 0] = fl.reshape(_PB)


def _rows_fl(xc, yc, zc, mins):
  return pl.pallas_call(
      _rows_body,
      grid=(_B, _NBK),
      in_specs=[pl.BlockSpec((1, 1, _PB),
                             lambda b, j: (b * _NBK + j, 0, 0))] * 3
      + [pl.BlockSpec((1, 3, 128), lambda b, j: (b, 0, 0))],
      out_specs=[
          pl.BlockSpec((1, _NROW, _PB // 128, 128),
                       lambda b, j: (b * _NBK + j, 0, 0, 0)),
          pl.BlockSpec((1, _NROW, _PB // 128, 128),
                       lambda b, j: (b * _NBK + j, 0, 0, 0)),
          pl.BlockSpec((1, 1, _PB), lambda b, j: (b * _NBK + j, 0, 0)),
          pl.BlockSpec((1, 1, _PB), lambda b, j: (b * _NBK + j, 0, 0)),
      ],
      out_shape=[
          jax.ShapeDtypeStruct((_B * _NBK, _NROW, _PB // 128, 128),
                               jnp.float32),
          jax.ShapeDtypeStruct((_B * _NBK, _NROW, _PB // 128, 128),
                               jnp.float32),
          jax.ShapeDtypeStruct((_B * _NBK, 1, _PB), jnp.int32),
          jax.ShapeDtypeStruct((_B * _NBK, 1, _PB), jnp.int32),
      ],
      compiler_params=pltpu.CompilerParams(
          dimension_semantics=("parallel", "parallel")),
  )(xc, yc, zc, mins)


_CP = 1024
_NFC = _NPT // _CP
_TAIL = _NPT - _NFC * _CP


def _sc_body(rowsa_hbm, rowsb_hbm, fl2_hbm, fl1_hbm, gidx_hbm, zr_hbm,
             rawa_hbm, rawb_hbm,
             table_sh, fb0, fb1, rb0, rb1, sidxv, sflv, zrows, grows,
             sem, cs0, cs1, as0, as1):
  c = lax.axis_index("c")
  s = lax.axis_index("s")
  pltpu.sync_copy(zr_hbm, zrows)

  def start_copies(rows_hbm, k, fb, rb, cs):
    pass

  def do_all(rows_hbm, raw_hbm):
    for b in range(_B):
      rowbase = b * _NPAD + s * _NPT
      flrow0 = rowbase // 128
      obase = b * _D + s * _DS

      def cps(k, fb, rb, cs):
        pltpu.async_copy(fl2_hbm.at[pl.ds(flrow0 + k * (_CP // 128),
                                          _CP // 128)], fb, cs)
        pltpu.async_copy(rows_hbm.at[pl.ds(rowbase + k * _CP, _CP)], rb, cs)

      def wait_cps(fb, rb, cs):
        pltpu.make_async_copy(fl2_hbm.at[pl.ds(0, _CP // 128)], fb, cs).wait()
        pltpu.make_async_copy(rows_hbm.at[pl.ds(0, _CP)], rb, cs).wait()

      def adds(fb, rb, asem):
        for q in range(_CP // 128):
          pltpu.async_copy(rb.at[pl.ds(q * 128, 128)],
                           table_sh.at[fb.at[q]], asem, add=True)

      def wait_adds(fb, rb, asem):
        for q in range(_CP // 128):
          pltpu.make_async_copy(rb.at[pl.ds(q * 128, 128)],
                                table_sh.at[fb.at[q]], asem).wait()

      pltpu.sync_copy(gidx_hbm.at[pl.ds(obase, _DS)], sidxv)
      pltpu.async_copy(fl1_hbm.at[sidxv], sflv, sem).wait()
      plsc.subcore_barrier()
      pltpu.async_copy(zrows, table_sh.at[sflv], sem).wait()
      plsc.subcore_barrier()

      cps(0, fb0, rb0, cs0)

      def pair(i, _):
        wait_cps(fb0, rb0, cs0)

        @pl.when(i > 0)
        def _():
          wait_adds(fb1, rb1, as1)

        cps(2 * i + 1, fb1, rb1, cs1)
        adds(fb0, rb0, as0)
        wait_cps(fb1, rb1, cs1)
        wait_adds(fb0, rb0, as0)

        @pl.when(i < _NFC // 2 - 1)
        def _():
          cps(2 * i + 2, fb0, rb0, cs0)

        adds(fb1, rb1, as1)
        return 0

      lax.fori_loop(0, _NFC // 2, pair, 0)
      wait_adds(fb1, rb1, as1)
      toff = rowbase + _NFC * _CP
      pltpu.sync_copy(fl2_hbm.at[pl.ds(flrow0 + _NFC * (_CP // 128),
                                       _TAIL // 128)],
                      fb0.at[pl.ds(0, _TAIL // 128)])
      pltpu.sync_copy(rows_hbm.at[pl.ds(toff, _TAIL)],
                      rb0.at[pl.ds(0, _TAIL)])
      for q in range(_TAIL // 128):
        pltpu.sync_copy(rb0.at[pl.ds(q * 128, 128)],
                        table_sh.at[fb0.at[q]], add=True)

      plsc.subcore_barrier()
      pltpu.async_copy(table_sh.at[sflv], grows, sem).wait()
      pltpu.sync_copy(grows, raw_hbm.at[pl.ds(obase, _DS)])

  @pl.when(c == 0)
  def _():
    do_all(rowsa_hbm, rawa_hbm)

  @pl.when(c == 1)
  def _():
    do_all(rowsb_hbm, rawb_hbm)


def _sc_bin(rowsa, rowsb, fl2d, fl1d, gidx):
  zr = jnp.zeros((_DS, _NROW), jnp.float32)
  run = pl.kernel(
      _sc_body,
      out_type=(jax.ShapeDtypeStruct((_B * _D, _NROW), jnp.float32),
                jax.ShapeDtypeStruct((_B * _D, _NROW), jnp.float32)),
      mesh=plsc.VectorSubcoreMesh(core_axis_name="c", subcore_axis_name="s",
                                  num_cores=_NC, num_subcores=_NS),
      scratch_types=[
          pltpu.VMEM_SHARED((_V, _NROW), jnp.float32),
          pltpu.VMEM((_CP // 128, 128), jnp.int32),
          pltpu.VMEM((_CP // 128, 128), jnp.int32),
          pltpu.VMEM((_CP, _NROW), jnp.float32),
          pltpu.VMEM((_CP, _NROW), jnp.float32),
          pltpu.VMEM((_DS,), jnp.int32),
          pltpu.VMEM((_DS,), jnp.int32),
          pltpu.VMEM((_DS, _NROW), jnp.float32),
          pltpu.VMEM((_DS, _NROW), jnp.float32),
          pltpu.SemaphoreType.DMA,
          pltpu.SemaphoreType.DMA,
          pltpu.SemaphoreType.DMA,
          pltpu.SemaphoreType.DMA,
          pltpu.SemaphoreType.DMA,
      ],
      compiler_params=pltpu.CompilerParams(use_tc_tiling_on_sc=False),
  )
  return run(rowsa, rowsb, fl2d, fl1d, gidx, zr)


def _fin_body(ra_ref, rb_ref, o_ref):
  cnt = ra_ref[0, 0]
  safe = jnp.maximum(cnt, 1.0)
  mx = ra_ref[0, 1] / safe
  my = ra_ref[0, 2] / safe
  mz = ra_ref[0, 3] / safe
  cxx = ra_ref[0, 4] / safe - mx * mx
  cxy = ra_ref[0, 5] / safe - mx * my
  cxz = ra_ref[0, 6] / safe - mx * mz
  cyy = ra_ref[0, 7] / safe - my * my
  cyz = rb_ref[0, 0] / safe - my * mz
  czz = rb_ref[0, 1] / safe - mz * mz
  o_ref[0] = jnp.stack(
      [mx, my, mz, cxx, cxy, cxz, cxy, cyy, cyz, cxz, cyz, czz])


def _finalize(rawa_t, rawb_t):
  return pl.pallas_call(
      _fin_body,
      grid=(_B,),
      in_specs=[pl.BlockSpec((1, _NROW, _D), lambda b: (b, 0, 0))] * 2,
      out_specs=pl.BlockSpec((1, 12, _D), lambda b: (b, 0, 0)),
      out_shape=jax.ShapeDtypeStruct((_B, 12, _D), jnp.float32),
  )(rawa_t, rawb_t)


@jax.jit
def kernel(x):
  B, N, _ = x.shape
  skey = jax.random.key(42)
  sampled_idx = jax.random.randint(skey, (B, _D), 0, N)
  gidx = (sampled_idx.astype(jnp.int32)
          + jnp.arange(B, dtype=jnp.int32)[:, None] * _NPAD).reshape(B * _D)

  xt = x.transpose(0, 2, 1)
  xt = jnp.pad(xt, ((0, 0), (0, 0), (0, _NPAD - N)),
               constant_values=_PAD_VAL)
  xc = xt[:, 0].reshape(_B * _NBK, 1, _PB)
  yc = xt[:, 1].reshape(_B * _NBK, 1, _PB)
  zc = xt[:, 2].reshape(_B * _NBK, 1, _PB)

  mins = _mins128(xc, yc, zc)
  rowsa, rowsb, fl, fl1 = _rows_fl(xc, yc, zc, mins)
  rowsa = rowsa.transpose(0, 2, 3, 1).reshape(B * _NPAD, _NROW)
  rowsb = rowsb.transpose(0, 2, 3, 1).reshape(B * _NPAD, _NROW)
  rawa, rawb = _sc_bin(rowsa, rowsb,
                       fl.reshape(B * _NPAD // 128, 128),
                       fl1.reshape(B * _NPAD), gidx)
  out = _finalize(rawa.reshape(B, _D, _NROW).transpose(0, 2, 1),
                  rawb.reshape(B, _D, _NROW).transpose(0, 2, 1))
  return out.transpose(0, 2, 1)


# speedup vs baseline: 2.8886x; 1.1595x over previous
import functools

import jax
import jax.numpy as jnp
from jax import lax
from jax.experimental import pallas as pl
from jax.experimental.pallas import tpu as pltpu
from jax.experimental.pallas import tpu_sc as plsc

_VS = 0.02
_G = 51
_V = _G ** 3
_D = 2048

_NC = 2
_NS = 16

_B = 4
_N = 200000
_NPT = ((_N + _NS * 128 - 1) // (_NS * 128)) * 128
_NPAD = _NS * _NPT
_DS = _D // _NS
_NROW = 8
_PB = 14336
_NBK = _NPAD // _PB
_PAD_VAL = 1e30


def _min_body(x_ref, y_ref, z_ref, o_ref):
  j = pl.program_id(1)

  @pl.when(j == 0)
  def _():
    o_ref[...] = jnp.full_like(o_ref, _PAD_VAL)

  def red(ref):
    return jnp.min(ref[0, 0].reshape(_PB // 128, 128), axis=0)

  m = jnp.stack([red(x_ref), red(y_ref), red(z_ref)])
  o_ref[...] = jnp.minimum(o_ref[...], m[None])


def _mins128(xc, yc, zc):
  return pl.pallas_call(
      _min_body,
      grid=(_B, _NBK),
      in_specs=[pl.BlockSpec((1, 1, _PB),
                             lambda b, j: (b * _NBK + j, 0, 0))] * 3,
      out_specs=pl.BlockSpec((1, 3, 128), lambda b, j: (b, 0, 0)),
      out_shape=jax.ShapeDtypeStruct((_B, 3, 128), jnp.float32),
      compiler_params=pltpu.CompilerParams(
          dimension_semantics=("arbitrary", "arbitrary")),
  )(xc, yc, zc)


def _rows_body(x_ref, y_ref, z_ref, m_ref, rowsa_ref, rowsb_ref, fl_ref,
               fl1_ref):
  j = pl.program_id(1)
  m0 = jnp.min(m_ref[0, 0])
  m1 = jnp.min(m_ref[0, 1])
  m2 = jnp.min(m_ref[0, 2])
  shape = (_PB // 128, 128)
  gidx = (j * _PB
          + lax.broadcasted_iota(jnp.int32, shape, 0) * 128
          + lax.broadcasted_iota(jnp.int32, shape, 1))
  valid = gidx < _N
  px = x_ref[0, 0].reshape(shape)
  py = y_ref[0, 0].reshape(shape)
  pz = z_ref[0, 0].reshape(shape)
  vs = jnp.float32(_VS)

  def vox(p, m):
    pm = jnp.where(valid, p, m)
    return jnp.clip(((pm - m) / vs).astype(jnp.int32), 0, _G - 1)

  fl = (vox(px, m0) * _G + vox(py, m1)) * _G + vox(pz, m2)
  px = jnp.where(valid, px, 0.0)
  py = jnp.where(valid, py, 0.0)
  pz = jnp.where(valid, pz, 0.0)
  cnt = valid.astype(jnp.float32)
  z0 = jnp.zeros_like(cnt)
  va = (cnt, px, py, pz, px * px, px * py, px * pz, py * py)
  vb = (py * pz, pz * pz, z0, z0, z0, z0, z0, z0)
  for v in range(_NROW):
    rowsa_ref[0, v] = va[v]
    rowsb_ref[0, v] = vb[v]
  fl_ref[0, 0] = fl.reshape(_PB)
  fl1_ref[0, 0] = fl.reshape(_PB)


def _rows_fl(xc, yc, zc, mins):
  return pl.pallas_call(
      _rows_body,
      grid=(_B, _NBK),
      in_specs=[pl.BlockSpec((1, 1, _PB),
                             lambda b, j: (b * _NBK + j, 0, 0))] * 3
      + [pl.BlockSpec((1, 3, 128), lambda b, j: (b, 0, 0))],
      out_specs=[
          pl.BlockSpec((1, _NROW, _PB // 128, 128),
                       lambda b, j: (b * _NBK + j, 0, 0, 0)),
          pl.BlockSpec((1, _NROW, _PB // 128, 128),
                       lambda b, j: (b * _NBK + j, 0, 0, 0)),
          pl.BlockSpec((1, 1, _PB), lambda b, j: (b * _NBK + j, 0, 0)),
          pl.BlockSpec((1, 1, _PB), lambda b, j: (b * _NBK + j, 0, 0)),
      ],
      out_shape=[
          jax.ShapeDtypeStruct((_B * _NBK, _NROW, _PB // 128, 128),
                               jnp.float32),
          jax.ShapeDtypeStruct((_B * _NBK, _NROW, _PB // 128, 128),
                               jnp.float32),
          jax.ShapeDtypeStruct((_B * _NBK, 1, _PB), jnp.int32),
          jax.ShapeDtypeStruct((_B * _NBK, 1, _PB), jnp.int32),
      ],
      compiler_params=pltpu.CompilerParams(
          dimension_semantics=("parallel", "parallel")),
  )(xc, yc, zc, mins)


_CP = 1024
_NFC = _NPT // _CP
_TAIL = _NPT - _NFC * _CP


def _sc_body(rowsa_hbm, rowsb_hbm, fl2_hbm, fl1_hbm, gidx_hbm, zr_hbm,
             rawa_hbm, rawb_hbm,
             table_sh, fb0, fb1, rb0, rb1, sidxv, sflv, zrows, grows,
             sem, cs0, cs1, as0, as1):
  c = lax.axis_index("c")
  s = lax.axis_index("s")
  pltpu.sync_copy(zr_hbm, zrows)

  def start_copies(rows_hbm, k, fb, rb, cs):
    pass

  def do_all(rows_hbm, raw_hbm):
    for b in range(_B):
      rowbase = b * _NPAD + s * _NPT
      flrow0 = rowbase // 128
      obase = b * _D + s * _DS

      def cps(k, fb, rb, cs):
        pltpu.async_copy(fl2_hbm.at[pl.ds(flrow0 + k * (_CP // 128),
                                          _CP // 128)], fb, cs)
        pltpu.async_copy(rows_hbm.at[pl.ds(rowbase + k * _CP, _CP)], rb, cs)

      def wait_cps(fb, rb, cs):
        pltpu.make_async_copy(fl2_hbm.at[pl.ds(0, _CP // 128)], fb, cs).wait()
        pltpu.make_async_copy(rows_hbm.at[pl.ds(0, _CP)], rb, cs).wait()

      def adds(fb, rb, asem):
        for q in range(_CP // 128):
          pltpu.async_copy(rb.at[pl.ds(q * 128, 128)],
                           table_sh.at[fb.at[q]], asem, add=True)

      def wait_adds(fb, rb, asem):
        for q in range(_CP // 128):
          pltpu.make_async_copy(rb.at[pl.ds(q * 128, 128)],
                                table_sh.at[fb.at[q]], asem).wait()

      pltpu.sync_copy(gidx_hbm.at[pl.ds(obase, _DS)], sidxv)
      pltpu.async_copy(fl1_hbm.at[sidxv], sflv, sem).wait()
      plsc.subcore_barrier()
      pltpu.async_copy(zrows, table_sh.at[sflv], sem).wait()
      plsc.subcore_barrier()

      cps(0, fb0, rb0, cs0)

      def pair(i, _):
        wait_cps(fb0, rb0, cs0)

        @pl.when(i > 0)
        def _():
          wait_adds(fb1, rb1, as1)

        cps(2 * i + 1, fb1, rb1, cs1)
        adds(fb0, rb0, as0)
        wait_cps(fb1, rb1, cs1)
        wait_adds(fb0, rb0, as0)

        @pl.when(i < _NFC // 2 - 1)
        def _():
          cps(2 * i + 2, fb0, rb0, cs0)

        adds(fb1, rb1, as1)
        return 0

      lax.fori_loop(0, _NFC // 2, pair, 0)
      wait_adds(fb1, rb1, as1)
      toff = rowbase + _NFC * _CP
      pltpu.sync_copy(fl2_hbm.at[pl.ds(flrow0 + _NFC * (_CP // 128),
                                       _TAIL // 128)],
                      fb0.at[pl.ds(0, _TAIL // 128)])
      pltpu.sync_copy(rows_hbm.at[pl.ds(toff, _TAIL)],
                      rb0.at[pl.ds(0, _TAIL)])
      for q in range(_TAIL // 128):
        pltpu.sync_copy(rb0.at[pl.ds(q * 128, 128)],
                        table_sh.at[fb0.at[q]], add=True)

      plsc.subcore_barrier()
      pltpu.async_copy(table_sh.at[sflv], grows, sem).wait()
      pltpu.sync_copy(grows, raw_hbm.at[pl.ds(obase, _DS)])

  @pl.when(c == 0)
  def _():
    do_all(rowsa_hbm, rawa_hbm)

  @pl.when(c == 1)
  def _():
    do_all(rowsb_hbm, rawb_hbm)


def _sc_bin(rowsa, rowsb, fl2d, fl1d, gidx):
  zr = jnp.zeros((_DS, _NROW), jnp.float32)
  run = pl.kernel(
      _sc_body,
      out_type=(jax.ShapeDtypeStruct((_B * _D, _NROW), jnp.float32),
                jax.ShapeDtypeStruct((_B * _D, _NROW), jnp.float32)),
      mesh=plsc.VectorSubcoreMesh(core_axis_name="c", subcore_axis_name="s",
                                  num_cores=_NC, num_subcores=_NS),
      scratch_types=[
          pltpu.VMEM_SHARED((_V, _NROW), jnp.float32),
          pltpu.VMEM((_CP // 128, 128), jnp.int32),
          pltpu.VMEM((_CP // 128, 128), jnp.int32),
          pltpu.VMEM((_CP, _NROW), jnp.float32),
          pltpu.VMEM((_CP, _NROW), jnp.float32),
          pltpu.VMEM((_DS,), jnp.int32),
          pltpu.VMEM((_DS,), jnp.int32),
          pltpu.VMEM((_DS, _NROW), jnp.float32),
          pltpu.VMEM((_DS, _NROW), jnp.float32),
          pltpu.SemaphoreType.DMA,
          pltpu.SemaphoreType.DMA,
          pltpu.SemaphoreType.DMA,
          pltpu.SemaphoreType.DMA,
          pltpu.SemaphoreType.DMA,
      ],
      compiler_params=pltpu.CompilerParams(use_tc_tiling_on_sc=False),
  )
  return run(rowsa, rowsb, fl2d, fl1d, gidx, zr)


def _fin_body(ra_ref, rb_ref, o_ref):
  cnt = ra_ref[0, 0]
  safe = jnp.maximum(cnt, 1.0)
  mx = ra_ref[0, 1] / safe
  my = ra_ref[0, 2] / safe
  mz = ra_ref[0, 3] / safe
  cxx = ra_ref[0, 4] / safe - mx * mx
  cxy = ra_ref[0, 5] / safe - mx * my
  cxz = ra_ref[0, 6] / safe - mx * mz
  cyy = ra_ref[0, 7] / safe - my * my
  cyz = rb_ref[0, 0] / safe - my * mz
  czz = rb_ref[0, 1] / safe - mz * mz
  o_ref[0] = jnp.stack(
      [mx, my, mz, cxx, cxy, cxz, cxy, cyy, cyz, cxz, cyz, czz])


def _finalize(rawa_t, rawb_t):
  return pl.pallas_call(
      _fin_body,
      grid=(_B,),
      in_specs=[pl.BlockSpec((1, _NROW, _D), lambda b: (b, 0, 0))] * 2,
      out_specs=pl.BlockSpec((1, 12, _D), lambda b: (b, 0, 0)),
      out_shape=jax.ShapeDtypeStruct((_B, 12, _D), jnp.float32),
  )(rawa_t, rawb_t)


@jax.jit
def kernel(x):
  B, N, _ = x.shape
  skey = jax.random.key(42)
  sampled_idx = jax.random.randint(skey, (B, _D), 0, N)
  gidx = (sampled_idx.astype(jnp.int32)
          + jnp.arange(B, dtype=jnp.int32)[:, None] * _NPAD).reshape(B * _D)

  xt = x.transpose(0, 2, 1)
  xt = jnp.pad(xt, ((0, 0), (0, 0), (0, _NPAD - N)),
               constant_values=_PAD_VAL)
  xc = xt[:, 0].reshape(_B * _NBK, 1, _PB)
  yc = xt[:, 1].reshape(_B * _NBK, 1, _PB)
  zc = xt[:, 2].reshape(_B * _NBK, 1, _PB)

  mins = _mins128(xc, yc, zc)
  rowsa, rowsb, fl, fl1 = _rows_fl(xc, yc, zc, mins)
  rowsa = rowsa.transpose(0, 2, 3, 1).reshape(B * _NPAD, _NROW)
  rowsb = rowsb.transpose(0, 2, 3, 1).reshape(B * _NPAD, _NROW)
  rawa, rawb = _sc_bin(rowsa, rowsb,
                       fl.reshape(B * _NPAD // 128, 128),
                       fl1.reshape(B * _NPAD), gidx)
  out = _finalize(rawa.reshape(B, _D, _NROW).transpose(0, 2, 1),
                  rawb.reshape(B, _D, _NROW).transpose(0, 2, 1))
  return out.transpose(0, 2, 1)
